# baseline (speedup 1.0000x reference)
import functools

import jax
import jax.numpy as jnp
from jax import lax
from jax.experimental import pallas as pl
from jax.experimental.pallas import tpu as pltpu
from jax.experimental.pallas import tpu_sc as plsc

_VOCAB = 100000
_EMB = 64
_MAX_NORM = 1.0
_B = 1024
_L = 20

_NC = 2
_NS = 16
_NW = _NC * _NS
_CHUNK = 128
_ROW = 2 * _EMB

_TB = 2048
_HB = 128
_VB = 2048


def _tp_body(tt_ref, o_ref):
    eye = (lax.broadcasted_iota(jnp.int32, (_EMB, _ROW), 0)
           == lax.broadcasted_iota(jnp.int32, (_EMB, _ROW), 1)
           ).astype(jnp.float32)
    o_ref[...] = lax.dot_general(
        tt_ref[...], eye,
        dimension_numbers=(((0,), (0,)), ((), ())),
        preferred_element_type=jnp.float32,
    )


def _sc_gather(table_hbm_shape_checked, idx3):
    n_chunks = idx3.shape[1]
    per_w = n_chunks * _CHUNK
    n = _NW * per_w
    mesh = plsc.VectorSubcoreMesh(core_axis_name="c", subcore_axis_name="s")

    @functools.partial(
        pl.kernel,
        mesh=mesh,
        out_type=jax.ShapeDtypeStruct((n, _ROW), jnp.float32),
        compiler_params=pltpu.CompilerParams(use_tc_tiling_on_sc=True),
        scratch_types=[
            pltpu.VMEM((n_chunks, _CHUNK), jnp.int32),
            pltpu.VMEM((per_w, _ROW), jnp.float32),
            pltpu.SemaphoreType.DMA,
        ],
    )
    def gather_kernel(table_hbm, idx_hbm, out_hbm, idx_v, rows_v, sem):
        wid = lax.axis_index("s") * _NC + lax.axis_index("c")
        pltpu.sync_copy(idx_hbm.at[wid], idx_v)
        copies = [
            pltpu.async_copy(
                table_hbm.at[idx_v.at[j]],
                rows_v.at[pl.ds(j * _CHUNK, _CHUNK)],
                sem,
            )
            for j in range(n_chunks)
        ]
        for c in copies:
            c.wait()
        pltpu.sync_copy(rows_v, out_hbm.at[pl.ds(wid * per_w, per_w)])

    return gather_kernel(table_hbm_shape_checked, idx3)


def _h_body(v_ref, grp_ref, o_ref):
    v = v_ref[:, :_EMB]
    ss = jnp.sum(v * v, axis=1, keepdims=True)
    norm = jnp.sqrt(ss)
    scale = jnp.where(norm > _MAX_NORM, _MAX_NORM / (norm + 1e-7), 1.0)
    sv = v * scale
    o_ref[...] = lax.dot_general(
        grp_ref[...], sv,
        dimension_numbers=(((1,), (0,)), ((), ())),
        precision=lax.Precision.HIGHEST,
        preferred_element_type=jnp.float32,
    ) * (1.0 / _L)


def _mm_body(wt_ref, h_ref, b_ref, o_ref):
    o_ref[...] = lax.dot_general(
        wt_ref[...], h_ref[...],
        dimension_numbers=(((0,), (1,)), ((), ())),
        preferred_element_type=jnp.float32,
    ) + jnp.transpose(b_ref[...])


def kernel(inputs, emb_table, W, b):
    table_rows = pl.pallas_call(
        _tp_body,
        grid=(pl.cdiv(_VOCAB, _TB),),
        in_specs=[pl.BlockSpec((_EMB, _TB), lambda i: (0, i))],
        out_specs=pl.BlockSpec((_TB, _ROW), lambda i: (i, 0)),
        out_shape=jax.ShapeDtypeStruct((_VOCAB, _ROW), jnp.float32),
    )(emb_table.T)

    idx3 = inputs.reshape(_NW, -1, _CHUNK)
    vecs = _sc_gather(table_rows, idx3)

    grp = (jnp.arange(_HB * _L, dtype=jnp.int32)[None, :] // _L
           == jnp.arange(_HB, dtype=jnp.int32)[:, None]).astype(jnp.float32)
    h = pl.pallas_call(
        _h_body,
        grid=(_B // _HB,),
        in_specs=[
            pl.BlockSpec((_HB * _L, _ROW), lambda i: (i, 0)),
            pl.BlockSpec((_HB, _HB * _L), lambda i: (0, 0)),
        ],
        out_specs=pl.BlockSpec((_HB, _EMB), lambda i: (i, 0)),
        out_shape=jax.ShapeDtypeStruct((_B, _EMB), jnp.float32),
    )(vecs, grp)

    b2 = b.reshape(1, _VOCAB)
    n_blocks = pl.cdiv(_VOCAB, _VB)
    logits_t = pl.pallas_call(
        _mm_body,
        grid=(n_blocks,),
        in_specs=[
            pl.BlockSpec((_EMB, _VB), lambda i: (0, i)),
            pl.BlockSpec((_B, _EMB), lambda i: (0, 0)),
            pl.BlockSpec((1, _VB), lambda i: (0, i)),
        ],
        out_specs=pl.BlockSpec((_VB, _B), lambda i: (i, 0)),
        out_shape=jax.ShapeDtypeStruct((_VOCAB, _B), jnp.float32),
    )(W.T, h, b2)
    return logits_t.T


# speedup vs baseline: 1.1066x; 1.1066x over previous
import functools

import jax
import jax.numpy as jnp
from jax import lax
from jax.experimental import pallas as pl
from jax.experimental.pallas import tpu as pltpu
from jax.experimental.pallas import tpu_sc as plsc

_VOCAB = 100000
_EMB = 64
_MAX_NORM = 1.0
_B = 1024
_L = 20

_NC = 2
_NS = 16
_NW = _NC * _NS
_CHUNK = 128
_ROW = 2 * _EMB

_TB = 8192
_HB = 128
_VB = 2048


def _tp_body(tt_ref, o_ref):
    eye = (lax.broadcasted_iota(jnp.int32, (_EMB, _ROW), 0)
           == lax.broadcasted_iota(jnp.int32, (_EMB, _ROW), 1)
           ).astype(jnp.float32)
    o_ref[...] = lax.dot_general(
        tt_ref[...], eye,
        dimension_numbers=(((0,), (0,)), ((), ())),
        preferred_element_type=jnp.float32,
    )


def _sc_gather(table_hbm_shape_checked, idx3):
    n_chunks = idx3.shape[1]
    per_w = n_chunks * _CHUNK
    n = _NW * per_w
    mesh = plsc.VectorSubcoreMesh(core_axis_name="c", subcore_axis_name="s")

    @functools.partial(
        pl.kernel,
        mesh=mesh,
        out_type=jax.ShapeDtypeStruct((n, _ROW), jnp.float32),
        compiler_params=pltpu.CompilerParams(use_tc_tiling_on_sc=True),
        scratch_types=[
            pltpu.VMEM((n_chunks, _CHUNK), jnp.int32),
            pltpu.VMEM((per_w, _ROW), jnp.float32),
            pltpu.SemaphoreType.DMA,
        ],
    )
    def gather_kernel(table_hbm, idx_hbm, out_hbm, idx_v, rows_v, sem):
        wid = lax.axis_index("s") * _NC + lax.axis_index("c")
        pltpu.sync_copy(idx_hbm.at[wid], idx_v)
        copies = [
            pltpu.async_copy(
                table_hbm.at[idx_v.at[j]],
                rows_v.at[pl.ds(j * _CHUNK, _CHUNK)],
                sem,
            )
            for j in range(n_chunks)
        ]
        for c in copies:
            c.wait()
        pltpu.sync_copy(rows_v, out_hbm.at[pl.ds(wid * per_w, per_w)])

    return gather_kernel(table_hbm_shape_checked, idx3)


def _h_body(v_ref, grp_ref, o_ref):
    v = v_ref[:, :_EMB]
    vv = v * v
    ones = jnp.full((_EMB, 128), 1.0, dtype=jnp.float32)
    ss = lax.dot_general(
        vv, ones,
        dimension_numbers=(((1,), (0,)), ((), ())),
        preferred_element_type=jnp.float32,
    )[:, :1]
    scale = jnp.minimum(jnp.float32(1.0), lax.rsqrt(ss) * _MAX_NORM)
    sv = v * scale
    o_ref[...] = lax.dot_general(
        grp_ref[...], sv,
        dimension_numbers=(((1,), (0,)), ((), ())),
        precision=lax.Precision.HIGHEST,
        preferred_element_type=jnp.float32,
    ) * (1.0 / _L)


def _mm_body(wt_ref, h_ref, b_ref, o_ref):
    o_ref[...] = lax.dot_general(
        wt_ref[...], h_ref[...],
        dimension_numbers=(((0,), (1,)), ((), ())),
        preferred_element_type=jnp.float32,
    ) + jnp.transpose(b_ref[...])


def kernel(inputs, emb_table, W, b):
    table_rows = pl.pallas_call(
        _tp_body,
        grid=(pl.cdiv(_VOCAB, _TB),),
        in_specs=[pl.BlockSpec((_EMB, _TB), lambda i: (0, i))],
        out_specs=pl.BlockSpec((_TB, _ROW), lambda i: (i, 0)),
        out_shape=jax.ShapeDtypeStruct((_VOCAB, _ROW), jnp.float32),
    )(emb_table.T)

    idx3 = inputs.reshape(_NW, -1, _CHUNK)
    vecs = _sc_gather(table_rows, idx3)

    grp = (jnp.arange(_HB * _L, dtype=jnp.int32)[None, :] // _L
           == jnp.arange(_HB, dtype=jnp.int32)[:, None]).astype(jnp.float32)
    h = pl.pallas_call(
        _h_body,
        grid=(_B // _HB,),
        in_specs=[
            pl.BlockSpec((_HB * _L, _ROW), lambda i: (i, 0)),
            pl.BlockSpec((_HB, _HB * _L), lambda i: (0, 0)),
        ],
        out_specs=pl.BlockSpec((_HB, _EMB), lambda i: (i, 0)),
        out_shape=jax.ShapeDtypeStruct((_B, _EMB), jnp.float32),
    )(vecs, grp)

    b2 = b.reshape(1, _VOCAB)
    n_blocks = pl.cdiv(_VOCAB, _VB)
    logits_t = pl.pallas_call(
        _mm_body,
        grid=(n_blocks,),
        in_specs=[
            pl.BlockSpec((_EMB, _VB), lambda i: (0, i)),
            pl.BlockSpec((_B, _EMB), lambda i: (0, 0)),
            pl.BlockSpec((1, _VB), lambda i: (0, i)),
        ],
        out_specs=pl.BlockSpec((_VB, _B), lambda i: (i, 0)),
        out_shape=jax.ShapeDtypeStruct((_VOCAB, _B), jnp.float32),
    )(W.T, h, b2)
    return logits_t.T


# speedup vs baseline: 1.1170x; 1.0094x over previous
import functools

import jax
import jax.numpy as jnp
from jax import lax
from jax.experimental import pallas as pl
from jax.experimental.pallas import tpu as pltpu
from jax.experimental.pallas import tpu_sc as plsc

_VOCAB = 100000
_EMB = 64
_MAX_NORM = 1.0
_B = 1024
_L = 20

_NC = 2
_NS = 16
_NW = _NC * _NS
_CHUNK = 128
_ROW = 2 * _EMB

_TB = 8192
_HB = 128
_VB = 4096


def _tp_body(tt_ref, o_ref):
    eye = (lax.broadcasted_iota(jnp.int32, (_EMB, _ROW), 0)
           == lax.broadcasted_iota(jnp.int32, (_EMB, _ROW), 1)
           ).astype(jnp.float32)
    o_ref[...] = lax.dot_general(
        tt_ref[...], eye,
        dimension_numbers=(((0,), (0,)), ((), ())),
        preferred_element_type=jnp.float32,
    )


def _sc_gather(table_hbm_shape_checked, idx3):
    n_chunks = idx3.shape[1]
    per_w = n_chunks * _CHUNK
    n = _NW * per_w
    mesh = plsc.VectorSubcoreMesh(core_axis_name="c", subcore_axis_name="s")

    @functools.partial(
        pl.kernel,
        mesh=mesh,
        out_type=jax.ShapeDtypeStruct((n, _ROW), jnp.float32),
        compiler_params=pltpu.CompilerParams(use_tc_tiling_on_sc=True),
        scratch_types=[
            pltpu.VMEM((n_chunks, _CHUNK), jnp.int32),
            pltpu.VMEM((per_w, _ROW), jnp.float32),
            pltpu.SemaphoreType.DMA,
        ],
    )
    def gather_kernel(table_hbm, idx_hbm, out_hbm, idx_v, rows_v, sem):
        wid = lax.axis_index("s") * _NC + lax.axis_index("c")
        pltpu.sync_copy(idx_hbm.at[wid], idx_v)
        copies = [
            pltpu.async_copy(
                table_hbm.at[idx_v.at[j]],
                rows_v.at[pl.ds(j * _CHUNK, _CHUNK)],
                sem,
            )
            for j in range(n_chunks)
        ]
        for c in copies:
            c.wait()
        pltpu.sync_copy(rows_v, out_hbm.at[pl.ds(wid * per_w, per_w)])

    return gather_kernel(table_hbm_shape_checked, idx3)


def _h_body(v_ref, grp_ref, o_ref):
    v = v_ref[:, :_EMB]
    vv = v * v
    ones = jnp.full((_EMB, 128), 1.0, dtype=jnp.float32)
    ss = lax.dot_general(
        vv, ones,
        dimension_numbers=(((1,), (0,)), ((), ())),
        preferred_element_type=jnp.float32,
    )[:, :1]
    scale = jnp.minimum(jnp.float32(1.0), lax.rsqrt(ss) * _MAX_NORM)
    sv = v * scale
    o_ref[...] = lax.dot_general(
        grp_ref[...], sv,
        dimension_numbers=(((1,), (0,)), ((), ())),
        precision=lax.Precision.HIGHEST,
        preferred_element_type=jnp.float32,
    ) * (1.0 / _L)


def _mm_body(wt_ref, h_ref, b_ref, o_ref):
    o_ref[...] = lax.dot_general(
        wt_ref[...], h_ref[...],
        dimension_numbers=(((0,), (1,)), ((), ())),
        preferred_element_type=jnp.float32,
    ) + jnp.transpose(b_ref[...])


def kernel(inputs, emb_table, W, b):
    table_rows = pl.pallas_call(
        _tp_body,
        grid=(pl.cdiv(_VOCAB, _TB),),
        in_specs=[pl.BlockSpec((_EMB, _TB), lambda i: (0, i))],
        out_specs=pl.BlockSpec((_TB, _ROW), lambda i: (i, 0)),
        out_shape=jax.ShapeDtypeStruct((_VOCAB, _ROW), jnp.float32),
    )(emb_table.T)

    idx3 = inputs.reshape(_NW, -1, _CHUNK)
    vecs = _sc_gather(table_rows, idx3)

    grp = (jnp.arange(_HB * _L, dtype=jnp.int32)[None, :] // _L
           == jnp.arange(_HB, dtype=jnp.int32)[:, None]).astype(jnp.float32)
    h = pl.pallas_call(
        _h_body,
        grid=(_B // _HB,),
        in_specs=[
            pl.BlockSpec((_HB * _L, _ROW), lambda i: (i, 0)),
            pl.BlockSpec((_HB, _HB * _L), lambda i: (0, 0)),
        ],
        out_specs=pl.BlockSpec((_HB, _EMB), lambda i: (i, 0)),
        out_shape=jax.ShapeDtypeStruct((_B, _EMB), jnp.float32),
    )(vecs, grp)

    b2 = b.reshape(1, _VOCAB)
    n_blocks = pl.cdiv(_VOCAB, _VB)
    logits_t = pl.pallas_call(
        _mm_body,
        grid=(n_blocks,),
        in_specs=[
            pl.BlockSpec((_EMB, _VB), lambda i: (0, i)),
            pl.BlockSpec((_B, _EMB), lambda i: (0, 0)),
            pl.BlockSpec((1, _VB), lambda i: (0, i)),
        ],
        out_specs=pl.BlockSpec((_VB, _B), lambda i: (i, 0)),
        out_shape=jax.ShapeDtypeStruct((_VOCAB, _B), jnp.float32),
    )(W.T, h, b2)
    return logits_t.T


# speedup vs baseline: 1.1399x; 1.0204x over previous
import functools

import jax
import jax.numpy as jnp
from jax import lax
from jax.experimental import pallas as pl
from jax.experimental.pallas import tpu as pltpu
from jax.experimental.pallas import tpu_sc as plsc

_VOCAB = 100000
_EMB = 64
_MAX_NORM = 1.0
_B = 1024
_L = 20

_NC = 2
_NS = 16
_NW = _NC * _NS
_CHUNK = 128
_ROW = 2 * _EMB

_TB = 16384
_HB = 64
_VB = 4096


def _tp_body(tt_ref, o_ref):
    eye = (lax.broadcasted_iota(jnp.int32, (_EMB, _ROW), 0)
           == lax.broadcasted_iota(jnp.int32, (_EMB, _ROW), 1)
           ).astype(jnp.float32)
    o_ref[...] = lax.dot_general(
        tt_ref[...], eye,
        dimension_numbers=(((0,), (0,)), ((), ())),
        preferred_element_type=jnp.float32,
    )


def _sc_gather(table_hbm_shape_checked, idx3):
    n_chunks = idx3.shape[1]
    per_w = n_chunks * _CHUNK
    n = _NW * per_w
    mesh = plsc.VectorSubcoreMesh(core_axis_name="c", subcore_axis_name="s")

    @functools.partial(
        pl.kernel,
        mesh=mesh,
        out_type=jax.ShapeDtypeStruct((n, _ROW), jnp.float32),
        compiler_params=pltpu.CompilerParams(use_tc_tiling_on_sc=True),
        scratch_types=[
            pltpu.VMEM((n_chunks, _CHUNK), jnp.int32),
            pltpu.VMEM((per_w, _ROW), jnp.float32),
            pltpu.SemaphoreType.DMA,
        ],
    )
    def gather_kernel(table_hbm, idx_hbm, out_hbm, idx_v, rows_v, sem):
        wid = lax.axis_index("s") * _NC + lax.axis_index("c")
        pltpu.sync_copy(idx_hbm.at[wid], idx_v)
        copies = [
            pltpu.async_copy(
                table_hbm.at[idx_v.at[j]],
                rows_v.at[pl.ds(j * _CHUNK, _CHUNK)],
                sem,
            )
            for j in range(n_chunks)
        ]
        for c in copies:
            c.wait()
        pltpu.sync_copy(rows_v, out_hbm.at[pl.ds(wid * per_w, per_w)])

    return gather_kernel(table_hbm_shape_checked, idx3)


def _h_body(v_ref, grp_ref, o_ref):
    v = v_ref[:, :_EMB]
    vv = v * v
    ones = jnp.full((_EMB, 128), 1.0, dtype=jnp.float32)
    ss = lax.dot_general(
        vv, ones,
        dimension_numbers=(((1,), (0,)), ((), ())),
        preferred_element_type=jnp.float32,
    )[:, :1]
    scale = jnp.minimum(jnp.float32(1.0), lax.rsqrt(ss) * _MAX_NORM)
    sv = v * scale
    o_ref[...] = lax.dot_general(
        grp_ref[...], sv,
        dimension_numbers=(((1,), (0,)), ((), ())),
        preferred_element_type=jnp.float32,
    ) * (1.0 / _L)


def _mm_body(wt_ref, h_ref, b_ref, o_ref):
    o_ref[...] = lax.dot_general(
        wt_ref[...], h_ref[...],
        dimension_numbers=(((0,), (1,)), ((), ())),
        preferred_element_type=jnp.float32,
    ) + jnp.transpose(b_ref[...])


def kernel(inputs, emb_table, W, b):
    table_rows = pl.pallas_call(
        _tp_body,
        grid=(pl.cdiv(_VOCAB, _TB),),
        in_specs=[pl.BlockSpec((_EMB, _TB), lambda i: (0, i))],
        out_specs=pl.BlockSpec((_TB, _ROW), lambda i: (i, 0)),
        out_shape=jax.ShapeDtypeStruct((_VOCAB, _ROW), jnp.float32),
    )(emb_table.T)

    idx3 = inputs.reshape(_NW, -1, _CHUNK)
    vecs = _sc_gather(table_rows, idx3)

    grp = (jnp.arange(_HB * _L, dtype=jnp.int32)[None, :] // _L
           == jnp.arange(_HB, dtype=jnp.int32)[:, None]).astype(jnp.float32)
    h = pl.pallas_call(
        _h_body,
        grid=(_B // _HB,),
        in_specs=[
            pl.BlockSpec((_HB * _L, _ROW), lambda i: (i, 0)),
            pl.BlockSpec((_HB, _HB * _L), lambda i: (0, 0)),
        ],
        out_specs=pl.BlockSpec((_HB, _EMB), lambda i: (i, 0)),
        out_shape=jax.ShapeDtypeStruct((_B, _EMB), jnp.float32),
    )(vecs, grp)

    b2 = b.reshape(1, _VOCAB)
    n_blocks = pl.cdiv(_VOCAB, _VB)
    logits_t = pl.pallas_call(
        _mm_body,
        grid=(n_blocks,),
        in_specs=[
            pl.BlockSpec((_EMB, _VB), lambda i: (0, i)),
            pl.BlockSpec((_B, _EMB), lambda i: (0, 0)),
            pl.BlockSpec((1, _VB), lambda i: (0, i)),
        ],
        out_specs=pl.BlockSpec((_VB, _B), lambda i: (i, 0)),
        out_shape=jax.ShapeDtypeStruct((_VOCAB, _B), jnp.float32),
    )(W.T, h, b2)
    return logits_t.T
